# Initial kernel scaffold; baseline (speedup 1.0000x reference)
import jax
import jax.numpy as jnp
from jax.experimental import pallas as pl


def kernel(x, fake_pos, pin_feature, W_conv, b_conv, W_pin, W1, b1, W2, b2, W3, b3, edge_index, macro_index):
    raise NotImplementedError("write your pallas kernel here")


# speedup vs baseline: 3.7439x; 3.7439x over previous
import functools

import jax
import jax.numpy as jnp
from jax import lax
from jax.experimental import pallas as pl
from jax.experimental.pallas import tpu as pltpu
from jax.experimental.pallas import tpu_sc as plsc

NN = 10000
EE = 320000
NH = 10000
DH = 128
NMAC = 512

NC, NS = 2, 16
NW = NC * NS
EW = EE // NW
K = 80
NCHUNK = EW // K
STRIPE = 624
TAIL0 = NS * STRIPE
TAILN = NH - TAIL0

def _leaky(v):
    return jnp.where(v >= 0, v, 0.1 * v)


def _striped(s, fn):
    fn(s * STRIPE, STRIPE)

    @pl.when(s == 0)
    def _():
        fn(TAIL0, TAILN)


def _tc_pre_body(x_ref, fp_ref, mac_ref, wc_ref, xt_ref, cnt_ref):
    x = x_ref[...]
    wc = wc_ref[...]
    xt = jnp.dot(x, wc[:DH, :], preferred_element_type=jnp.float32)
    fp = fp_ref[...]
    xt += fp[:, 0:1] * wc[DH:DH + 1, :] + fp[:, 1:2] * wc[DH + 1:DH + 2, :]
    ids = lax.broadcasted_iota(jnp.int32, (NN, 1), 0)
    cnt = jnp.zeros((NN, 1), jnp.float32)
    for j in range(NMAC // 128):
        mac = mac_ref[:, j * 128:(j + 1) * 128]
        cnt += jnp.sum((ids == mac).astype(jnp.float32), axis=1, keepdims=True)
    ismacro = (cnt > 0).astype(jnp.float32)
    xt += ismacro * wc[DH + 2:DH + 3, :]
    xt_ref[...] = xt
    cnt_ref[...] = cnt


_tc_pre = pl.pallas_call(
    _tc_pre_body,
    out_shape=(jax.ShapeDtypeStruct((NN, DH), jnp.float32),
               jax.ShapeDtypeStruct((NN, 1), jnp.float32)),
)


def _sc_hop1_body(xt_hbm, nidx_hbm, hidx_hbm, p0_hbm, p1_hbm, p2_hbm, p3_hbm,
                  z128_hbm, z1_hbm, ones_hbm,
                  e_out, p_out, b_out, d_out,
                  e_sh, p_sh0, p_sh1, p_sh2, p_sh3, b_sh, d_sh,
                  nbuf, hbuf, pbuf, rows, ones_v, sem):
    c = lax.axis_index("c")
    s = lax.axis_index("s")
    wid = s * NC + c
    _striped(s, lambda r, n: pltpu.sync_copy(
        z128_hbm.at[pl.ds(r, n), :], e_sh.at[pl.ds(r, n), :]))

    @pl.when(s == 0)
    def _():
        for a in (p_sh0, p_sh1, p_sh2, p_sh3, b_sh, d_sh):
            pltpu.sync_copy(z1_hbm, a)
    pltpu.sync_copy(ones_hbm, ones_v)
    plsc.subcore_barrier()

    base = wid * EW
    pins = (p0_hbm, p1_hbm, p2_hbm, p3_hbm)
    paccs = (p_sh0, p_sh1, p_sh2, p_sh3)

    def chunk(i, carry):
        off = base + i * K
        pltpu.sync_copy(nidx_hbm.at[pl.ds(off, K)], nbuf)
        pltpu.sync_copy(hidx_hbm.at[pl.ds(off, K)], hbuf)
        pltpu.async_copy(xt_hbm.at[nbuf], rows, sem).wait()
        pltpu.sync_copy(rows, e_sh.at[hbuf], add=True)
        for ph, pa in zip(pins, paccs):
            pltpu.sync_copy(ph.at[pl.ds(off, K)], pbuf)
            pltpu.sync_copy(pbuf, pa.at[hbuf], add=True)
        pltpu.sync_copy(ones_v, b_sh.at[hbuf], add=True)
        pltpu.sync_copy(ones_v, d_sh.at[nbuf], add=True)
        return carry

    lax.fori_loop(0, NCHUNK, chunk, 0)
    plsc.subcore_barrier()
    _striped(s, lambda r, n: pltpu.sync_copy(
        e_sh.at[pl.ds(r, n), :], e_out.at[c, pl.ds(r, n), :]))

    @pl.when(s == 0)
    def _():
        for k, pa in enumerate(paccs):
            pltpu.sync_copy(pa, p_out.at[c, k])
        pltpu.sync_copy(b_sh, b_out.at[c])
        pltpu.sync_copy(d_sh, d_out.at[c])


@functools.cache
def _get_sc_hop1():
    mesh = plsc.VectorSubcoreMesh(core_axis_name="c", subcore_axis_name="s",
                                  num_cores=NC, num_subcores=NS)
    return pl.kernel(
        _sc_hop1_body,
        out_type=(jax.ShapeDtypeStruct((NC, NH, DH), jnp.float32),
                  jax.ShapeDtypeStruct((NC, 4, NH), jnp.float32),
                  jax.ShapeDtypeStruct((NC, NH), jnp.float32),
                  jax.ShapeDtypeStruct((NC, NN), jnp.float32)),
        mesh=mesh,
        scratch_types=[
            pltpu.VMEM_SHARED((NH, DH), jnp.float32),
            pltpu.VMEM_SHARED((NH,), jnp.float32),
            pltpu.VMEM_SHARED((NH,), jnp.float32),
            pltpu.VMEM_SHARED((NH,), jnp.float32),
            pltpu.VMEM_SHARED((NH,), jnp.float32),
            pltpu.VMEM_SHARED((NH,), jnp.float32),
            pltpu.VMEM_SHARED((NN,), jnp.float32),
            pltpu.VMEM((K,), jnp.int32),
            pltpu.VMEM((K,), jnp.int32),
            pltpu.VMEM((K,), jnp.float32),
            pltpu.VMEM((K, DH), jnp.float32),
            pltpu.VMEM((K,), jnp.float32),
            pltpu.SemaphoreType.DMA,
        ],
    )


def _tc_mid_body(ep_ref, pp_ref, bp_ref, wp_ref, e_ref):
    p = ep_ref[0] + ep_ref[1]
    pf = pp_ref[0] + pp_ref[1]
    bd = bp_ref[0] + bp_ref[1]
    wp = wp_ref[...]
    for k in range(4):
        p += pf[:, k:k + 1] * wp[k:k + 1, :]
    e_ref[...] = p * (1.0 / jnp.maximum(bd, 1.0))


_tc_mid = pl.pallas_call(
    _tc_mid_body,
    out_shape=jax.ShapeDtypeStruct((NH, DH), jnp.float32),
)


def _sc_hop2_body(e_hbm, nidx_hbm, hidx_hbm, z128_hbm,
                  q_out,
                  q_sh, nbuf, hbuf, rows, sem):
    c = lax.axis_index("c")
    s = lax.axis_index("s")
    wid = s * NC + c
    _striped(s, lambda r, n: pltpu.sync_copy(
        z128_hbm.at[pl.ds(r, n), :], q_sh.at[pl.ds(r, n), :]))
    plsc.subcore_barrier()

    base = wid * EW

    def chunk(i, carry):
        off = base + i * K
        pltpu.sync_copy(nidx_hbm.at[pl.ds(off, K)], nbuf)
        pltpu.sync_copy(hidx_hbm.at[pl.ds(off, K)], hbuf)
        pltpu.async_copy(e_hbm.at[hbuf], rows, sem).wait()
        pltpu.sync_copy(rows, q_sh.at[nbuf], add=True)
        return carry

    lax.fori_loop(0, NCHUNK, chunk, 0)
    plsc.subcore_barrier()
    _striped(s, lambda r, n: pltpu.sync_copy(
        q_sh.at[pl.ds(r, n), :], q_out.at[c, pl.ds(r, n), :]))


@functools.cache
def _get_sc_hop2():
    mesh = plsc.VectorSubcoreMesh(core_axis_name="c", subcore_axis_name="s",
                                  num_cores=NC, num_subcores=NS)
    return pl.kernel(
        _sc_hop2_body,
        out_type=jax.ShapeDtypeStruct((NC, NN, DH), jnp.float32),
        mesh=mesh,
        scratch_types=[
            pltpu.VMEM_SHARED((NN, DH), jnp.float32),
            pltpu.VMEM((K,), jnp.int32),
            pltpu.VMEM((K,), jnp.int32),
            pltpu.VMEM((K, DH), jnp.float32),
            pltpu.SemaphoreType.DMA,
        ],
    )


def _tc_fin_body(qp_ref, dp_ref, cnt_ref, bc_ref, w1_ref, b1_ref, w2_ref,
                 b2_ref, w3_ref, b3_ref, out_ref):
    q = qp_ref[0] + qp_ref[1]
    dd = dp_ref[0] + dp_ref[1]
    xact = _leaky(q * (1.0 / jnp.maximum(dd, 1.0)) + bc_ref[...])
    s_all = jnp.sum(xact, axis=0, keepdims=True) * (1.0 / NN)
    cnt = cnt_ref[...]
    s_mac = lax.dot_general(cnt, xact, (((0,), (0,)), ((), ())),
                            preferred_element_type=jnp.float32) * (1.0 / NMAC)
    x1 = jnp.concatenate([s_mac, s_all], axis=1)
    h1 = _leaky(jnp.dot(x1, w1_ref[...], preferred_element_type=jnp.float32)
                + b1_ref[...])
    h2 = _leaky(jnp.dot(h1, w2_ref[...], preferred_element_type=jnp.float32)
                + b2_ref[...])
    out_ref[...] = (jnp.dot(h2, w3_ref[...], preferred_element_type=jnp.float32)
                    + b3_ref[...])


_tc_fin = pl.pallas_call(
    _tc_fin_body,
    out_shape=jax.ShapeDtypeStruct((1, 10), jnp.float32),
)


def kernel(x, fake_pos, pin_feature, W_conv, b_conv, W_pin,
           W1, b1, W2, b2, W3, b3, edge_index, macro_index):
    nidx = edge_index[0]
    hidx = edge_index[1]
    mac2d = macro_index.reshape(1, NMAC)

    xt, cnt = _tc_pre(x, fake_pos, mac2d, W_conv)

    z128 = jnp.zeros((NH, DH), jnp.float32)
    z1 = jnp.zeros((NH,), jnp.float32)
    ones_k = jnp.ones((K,), jnp.float32)
    pin_cols = [jnp.asarray(pin_feature[:, k]) for k in range(4)]

    e_p, p_p, b_p, d_p = _get_sc_hop1()(xt, nidx, hidx, *pin_cols,
                                        z128, z1, ones_k)
    e = _tc_mid(e_p, jnp.transpose(p_p, (0, 2, 1)), b_p[:, :, None], W_pin)
    q_p = _get_sc_hop2()(e, nidx, hidx, z128)
    out = _tc_fin(q_p, d_p[:, :, None], cnt, b_conv.reshape(1, DH),
                  W1, b1.reshape(1, DH), W2, b2.reshape(1, 64),
                  W3, b3.reshape(1, 10))
    return out


# speedup vs baseline: 9.6428x; 2.5756x over previous
import functools

import jax
import jax.numpy as jnp
from jax import lax
from jax.experimental import pallas as pl
from jax.experimental.pallas import tpu as pltpu
from jax.experimental.pallas import tpu_sc as plsc

NN = 10000
EE = 320000
NH = 10000
DH = 128
NMAC = 512

NC, NS = 2, 16
NW = NC * NS
EW = EE // NW
K = 80
NCHUNK = EW // K
NSET = 3
STRIPE = 624
TAIL0 = NS * STRIPE
TAILN = NH - TAIL0


def _leaky(v):
    return jnp.where(v >= 0, v, 0.1 * v)


def _striped(s, fn):
    fn(s * STRIPE, STRIPE)

    @pl.when(s == 0)
    def _():
        fn(TAIL0, TAILN)


def _tc_pre_body(x_ref, fp_ref, mac_ref, wc_ref, xt_ref, cnt_ref):
    x = x_ref[...]
    wc = wc_ref[...]
    xt = jnp.dot(x, wc[:DH, :], preferred_element_type=jnp.float32)
    fp = fp_ref[...]
    xt += fp[:, 0:1] * wc[DH:DH + 1, :] + fp[:, 1:2] * wc[DH + 1:DH + 2, :]
    ids = lax.broadcasted_iota(jnp.int32, (NN, 1), 0)
    cnt = jnp.zeros((NN, 1), jnp.float32)
    for j in range(NMAC // 128):
        mac = mac_ref[:, j * 128:(j + 1) * 128]
        cnt += jnp.sum((ids == mac).astype(jnp.float32), axis=1, keepdims=True)
    ismacro = (cnt > 0).astype(jnp.float32)
    xt += ismacro * wc[DH + 2:DH + 3, :]
    xt_ref[...] = xt
    cnt_ref[...] = cnt


_tc_pre = pl.pallas_call(
    _tc_pre_body,
    out_shape=(jax.ShapeDtypeStruct((NN, DH), jnp.float32),
               jax.ShapeDtypeStruct((NN, 1), jnp.float32)),
)


def _pipeline(load, lwt, gst, gwt, sst, swt, sms, smw):
    def prime(i, u):
        load(i, u)
        lwt(i, u)
        gst(i, u)

    prime(0, 0)
    prime(1, 1)

    def step(j, carry):
        for u in range(NSET):
            i = NSET * j + u
            gwt(i, u)
            sst(i, u)
            sms(i, u)
            if u == 0:
                @pl.when(j > 0)
                def _():
                    swt(i - 1, NSET - 1)
                    smw(i - 1, NSET - 1)
            else:
                swt(i - 1, u - 1)
                smw(i - 1, u - 1)
            un = (u + 2) % NSET
            load(i + 2, un)
            lwt(i + 2, un)
            gst(i + 2, un)
        return carry

    lax.fori_loop(0, (NCHUNK - 2) // NSET, step, 0)
    ia, ib = NCHUNK - 2, NCHUNK - 1
    gwt(ia, ia % NSET)
    sst(ia, ia % NSET)
    sms(ia, ia % NSET)
    swt(ia - 1, (ia - 1) % NSET)
    smw(ia - 1, (ia - 1) % NSET)
    gwt(ib, ib % NSET)
    sst(ib, ib % NSET)
    sms(ib, ib % NSET)
    swt(ia, ia % NSET)
    smw(ia, ia % NSET)
    swt(ib, ib % NSET)
    smw(ib, ib % NSET)


def _sc_hop1_body(xt_hbm, nidx_hbm, hidx_hbm, pins_hbm, z_hbm, z1_hbm,
                  ones_hbm,
                  e_out, p_out, b_out,
                  e_sh, p_sh0, p_sh1, p_sh2, p_sh3, b_sh,
                  nb0, nb1, nb2, hb0, hb1, hb2, pb0, pb1, pb2,
                  rw0, rw1, rw2, ones_v,
                  l0, l1, l2, g0, g1, g2, s0, s1, s2, t0, t1, t2):
    c = lax.axis_index("c")
    s = lax.axis_index("s")
    wid = s * NC + c
    _striped(s, lambda r, n: pltpu.sync_copy(
        z_hbm.at[pl.ds(r, n), :], e_sh.at[pl.ds(r, n), :]))

    @pl.when(s == 0)
    def _():
        for a in (p_sh0, p_sh1, p_sh2, p_sh3, b_sh):
            pltpu.sync_copy(z1_hbm, a)
    pltpu.sync_copy(ones_hbm, ones_v)
    plsc.subcore_barrier()

    nbufs = (nb0, nb1, nb2)
    hbufs = (hb0, hb1, hb2)
    pbufs = (pb0, pb1, pb2)
    rows = (rw0, rw1, rw2)
    lsem = (l0, l1, l2)
    gsem = (g0, g1, g2)
    ssem = (s0, s1, s2)
    tsem = (t0, t1, t2)
    paccs = (p_sh0, p_sh1, p_sh2, p_sh3)
    base = wid * EW

    def load(i, u):
        off = base + i * K
        pltpu.async_copy(nidx_hbm.at[pl.ds(off, K)], nbufs[u], lsem[u])
        pltpu.async_copy(hidx_hbm.at[pl.ds(off, K)], hbufs[u], lsem[u])
        pltpu.async_copy(pins_hbm.at[wid, i], pbufs[u], lsem[u])

    def lwt(i, u):
        off = base + i * K
        pltpu.make_async_copy(nidx_hbm.at[pl.ds(off, K)], nbufs[u],
                              lsem[u]).wait()
        pltpu.make_async_copy(hidx_hbm.at[pl.ds(off, K)], hbufs[u],
                              lsem[u]).wait()
        pltpu.make_async_copy(pins_hbm.at[wid, i], pbufs[u], lsem[u]).wait()

    def gst(i, u):
        pltpu.async_copy(xt_hbm.at[nbufs[u]], rows[u], gsem[u])

    def gwt(i, u):
        pltpu.make_async_copy(xt_hbm.at[nbufs[u]], rows[u], gsem[u]).wait()

    def sst(i, u):
        pltpu.async_copy(rows[u], e_sh.at[hbufs[u]], ssem[u], add=True)

    def swt(i, u):
        pltpu.make_async_copy(rows[u], e_sh.at[hbufs[u]], ssem[u]).wait()

    def sms(i, u):
        for cc in range(4):
            pltpu.async_copy(pbufs[u].at[cc], paccs[cc].at[hbufs[u]],
                             tsem[u], add=True)
        pltpu.async_copy(ones_v, b_sh.at[hbufs[u]], tsem[u], add=True)

    def smw(i, u):
        for cc in range(4):
            pltpu.make_async_copy(pbufs[u].at[cc], paccs[cc].at[hbufs[u]],
                                  tsem[u]).wait()
        pltpu.make_async_copy(ones_v, b_sh.at[hbufs[u]], tsem[u]).wait()

    _pipeline(load, lwt, gst, gwt, sst, swt, sms, smw)
    plsc.subcore_barrier()
    _striped(s, lambda r, n: pltpu.sync_copy(
        e_sh.at[pl.ds(r, n), :], e_out.at[c, pl.ds(r, n), :]))

    @pl.when(s == 0)
    def _():
        for k, pa in enumerate(paccs):
            pltpu.sync_copy(pa, p_out.at[c, k])
        pltpu.sync_copy(b_sh, b_out.at[c])


def _sc_hop2_body(e_hbm, nidx_hbm, hidx_hbm, z_hbm, z1_hbm, ones_hbm,
                  q_out, d_out,
                  q_sh, d_sh,
                  gb0, gb1, gb2, sb0, sb1, sb2,
                  rw0, rw1, rw2, ones_v,
                  l0, l1, l2, g0, g1, g2, s0, s1, s2, t0, t1, t2):
    c = lax.axis_index("c")
    s = lax.axis_index("s")
    wid = s * NC + c
    _striped(s, lambda r, n: pltpu.sync_copy(
        z_hbm.at[pl.ds(r, n), :], q_sh.at[pl.ds(r, n), :]))

    @pl.when(s == 0)
    def _():
        pltpu.sync_copy(z1_hbm, d_sh)
    pltpu.sync_copy(ones_hbm, ones_v)
    plsc.subcore_barrier()

    gbufs = (gb0, gb1, gb2)
    sbufs = (sb0, sb1, sb2)
    rows = (rw0, rw1, rw2)
    lsem = (l0, l1, l2)
    gsem = (g0, g1, g2)
    ssem = (s0, s1, s2)
    tsem = (t0, t1, t2)
    base = wid * EW

    def load(i, u):
        off = base + i * K
        pltpu.async_copy(hidx_hbm.at[pl.ds(off, K)], gbufs[u], lsem[u])
        pltpu.async_copy(nidx_hbm.at[pl.ds(off, K)], sbufs[u], lsem[u])

    def lwt(i, u):
        off = base + i * K
        pltpu.make_async_copy(hidx_hbm.at[pl.ds(off, K)], gbufs[u],
                              lsem[u]).wait()
        pltpu.make_async_copy(nidx_hbm.at[pl.ds(off, K)], sbufs[u],
                              lsem[u]).wait()

    def gst(i, u):
        pltpu.async_copy(e_hbm.at[gbufs[u]], rows[u], gsem[u])

    def gwt(i, u):
        pltpu.make_async_copy(e_hbm.at[gbufs[u]], rows[u], gsem[u]).wait()

    def sst(i, u):
        pltpu.async_copy(rows[u], q_sh.at[sbufs[u]], ssem[u], add=True)

    def swt(i, u):
        pltpu.make_async_copy(rows[u], q_sh.at[sbufs[u]], ssem[u]).wait()

    def sms(i, u):
        pltpu.async_copy(ones_v, d_sh.at[sbufs[u]], tsem[u], add=True)

    def smw(i, u):
        pltpu.make_async_copy(ones_v, d_sh.at[sbufs[u]], tsem[u]).wait()

    _pipeline(load, lwt, gst, gwt, sst, swt, sms, smw)
    plsc.subcore_barrier()
    _striped(s, lambda r, n: pltpu.sync_copy(
        q_sh.at[pl.ds(r, n), :], q_out.at[c, pl.ds(r, n), :]))

    @pl.when(s == 0)
    def _():
        pltpu.sync_copy(d_sh, d_out.at[c])


def _mesh():
    return plsc.VectorSubcoreMesh(core_axis_name="c", subcore_axis_name="s",
                                  num_cores=NC, num_subcores=NS)


@functools.cache
def _get_sc_hop1():
    return pl.kernel(
        _sc_hop1_body,
        out_type=(jax.ShapeDtypeStruct((NC, NH, DH), jnp.float32),
                  jax.ShapeDtypeStruct((NC, 4, NH), jnp.float32),
                  jax.ShapeDtypeStruct((NC, NH), jnp.float32)),
        mesh=_mesh(),
        scratch_types=(
            [pltpu.VMEM_SHARED((NH, DH), jnp.float32)] +
            [pltpu.VMEM_SHARED((NH,), jnp.float32)] * 5 +
            [pltpu.VMEM((K,), jnp.int32)] * 6 +
            [pltpu.VMEM((4, K), jnp.float32)] * 3 +
            [pltpu.VMEM((K, DH), jnp.float32)] * 3 +
            [pltpu.VMEM((K,), jnp.float32)] +
            [pltpu.SemaphoreType.DMA] * 12
        ),
    )


@functools.cache
def _get_sc_hop2():
    return pl.kernel(
        _sc_hop2_body,
        out_type=(jax.ShapeDtypeStruct((NC, NN, DH), jnp.float32),
                  jax.ShapeDtypeStruct((NC, NN), jnp.float32)),
        mesh=_mesh(),
        scratch_types=(
            [pltpu.VMEM_SHARED((NN, DH), jnp.float32)] +
            [pltpu.VMEM_SHARED((NN,), jnp.float32)] +
            [pltpu.VMEM((K,), jnp.int32)] * 6 +
            [pltpu.VMEM((K, DH), jnp.float32)] * 3 +
            [pltpu.VMEM((K,), jnp.float32)] +
            [pltpu.SemaphoreType.DMA] * 12
        ),
    )


def _tc_mid_body(ep_ref, pp_ref, bp_ref, wp_ref, e_ref):
    p = ep_ref[0] + ep_ref[1]
    pf = pp_ref[0] + pp_ref[1]
    bd = bp_ref[0] + bp_ref[1]
    wp = wp_ref[...]
    for k in range(4):
        p += pf[:, k:k + 1] * wp[k:k + 1, :]
    e_ref[...] = p * (1.0 / jnp.maximum(bd, 1.0))


_tc_mid = pl.pallas_call(
    _tc_mid_body,
    out_shape=jax.ShapeDtypeStruct((NH, DH), jnp.float32),
)


def _tc_fin_body(qp_ref, dp_ref, cnt_ref, bc_ref, w1_ref, b1_ref, w2_ref,
                 b2_ref, w3_ref, b3_ref, out_ref):
    q = qp_ref[0] + qp_ref[1]
    dd = dp_ref[0] + dp_ref[1]
    xact = _leaky(q * (1.0 / jnp.maximum(dd, 1.0)) + bc_ref[...])
    s_all = jnp.sum(xact, axis=0, keepdims=True) * (1.0 / NN)
    cnt = cnt_ref[...]
    s_mac = lax.dot_general(cnt, xact, (((0,), (0,)), ((), ())),
                            preferred_element_type=jnp.float32) * (1.0 / NMAC)
    x1 = jnp.concatenate([s_mac, s_all], axis=1)
    h1 = _leaky(jnp.dot(x1, w1_ref[...], preferred_element_type=jnp.float32)
                + b1_ref[...])
    h2 = _leaky(jnp.dot(h1, w2_ref[...], preferred_element_type=jnp.float32)
                + b2_ref[...])
    out_ref[...] = (jnp.dot(h2, w3_ref[...], preferred_element_type=jnp.float32)
                    + b3_ref[...])


_tc_fin = pl.pallas_call(
    _tc_fin_body,
    out_shape=jax.ShapeDtypeStruct((1, 10), jnp.float32),
)


def kernel(x, fake_pos, pin_feature, W_conv, b_conv, W_pin,
           W1, b1, W2, b2, W3, b3, edge_index, macro_index):
    nidx = edge_index[0]
    hidx = edge_index[1]
    pins = jnp.transpose(pin_feature.reshape(NW, NCHUNK, K, 4), (0, 1, 3, 2))
    mac2d = macro_index.reshape(1, NMAC)

    xt, cnt = _tc_pre(x, fake_pos, mac2d, W_conv)

    z = jnp.zeros((NH, DH), jnp.float32)
    z1 = jnp.zeros((NH,), jnp.float32)
    ones_k = jnp.ones((K,), jnp.float32)

    e_p, p_p, b_p = _get_sc_hop1()(xt, nidx, hidx, pins, z, z1, ones_k)
    e = _tc_mid(e_p, jnp.transpose(p_p, (0, 2, 1)), b_p[:, :, None], W_pin)
    q_p, d_p = _get_sc_hop2()(e, nidx, hidx, z, z1, ones_k)
    out = _tc_fin(q_p, d_p[:, :, None], cnt, b_conv.reshape(1, DH),
                  W1, b1.reshape(1, DH), W2, b2.reshape(1, 64),
                  W3, b3.reshape(1, 10))
    return out
